# baseline (speedup 1.0000x reference)
import functools

import jax
import jax.numpy as jnp
from jax import lax
from jax.experimental import pallas as pl
from jax.experimental.pallas import tpu as pltpu
from jax.experimental.pallas import tpu_sc as plsc

_PAD_IDX = 1


def _sc_gather(emb_weight, idx_flat, n_rows, d):
    info = plsc.get_sparse_core_info()
    nw = 1 * info.num_subcores
    rows_per_w = n_rows // nw

    mesh = plsc.VectorSubcoreMesh(core_axis_name="c", subcore_axis_name="s", num_cores=1)

    @functools.partial(
        pl.kernel,
        mesh=mesh,
        out_type=jax.ShapeDtypeStruct((n_rows, d), jnp.float32),
        scratch_types=[
            pltpu.VMEM((rows_per_w,), jnp.int32),
            pltpu.VMEM((rows_per_w, d), jnp.float32),
            pltpu.SemaphoreType.DMA,
        ],
    )
    def gather_kernel(emb_hbm, idx_hbm, out_hbm, idx_v, rows_v, sem):
        wid = lax.axis_index("s")
        base = wid * rows_per_w
        pltpu.sync_copy(idx_hbm.at[pl.ds(base, rows_per_w)], idx_v)
        pltpu.async_copy(emb_hbm.at[idx_v], rows_v, sem).wait()
        pltpu.sync_copy(rows_v, out_hbm.at[pl.ds(base, rows_per_w)])

    return gather_kernel(emb_weight, idx_flat)


_ND = 2


def _merge_body(p_ref, x_ref, o_ref):
    p = p_ref.shape[1]
    t = x_ref.shape[1]
    o_ref[0, 0:p] = p_ref[0]
    o_ref[0, p : p + t] = x_ref[0]


def kernel(x, encoder_padding_mask, src_prompt, source_prompt_length_padding_mask, emb_weight):
    b, t, d = x.shape
    p = src_prompt.shape[1]

    idx = jnp.where(source_prompt_length_padding_mask, _PAD_IDX, src_prompt)
    idx_flat = idx.astype(jnp.int32).reshape(b * p)

    prompt_rows = _sc_gather(emb_weight, idx_flat, b * p, d)
    prompt_emb = prompt_rows.reshape(b, p, d)

    dsub = d // _ND
    out = pl.pallas_call(
        _merge_body,
        grid=(b, _ND),
        in_specs=[
            pl.BlockSpec((1, p, dsub), lambda bi, jd: (bi, 0, jd)),
            pl.BlockSpec((1, t, dsub), lambda bi, jd: (bi, 0, jd)),
        ],
        out_specs=pl.BlockSpec((1, p + t, dsub), lambda bi, jd: (bi, 0, jd)),
        out_shape=jax.ShapeDtypeStruct((b, p + t, d), x.dtype),
        compiler_params=pltpu.CompilerParams(
            dimension_semantics=("arbitrary", "arbitrary"),
        ),
    )(prompt_emb, x)

    out_padding_mask = jnp.concatenate(
        [source_prompt_length_padding_mask, encoder_padding_mask], axis=1
    )
    return out, out_padding_mask
